# baseline (speedup 1.0000x reference)
import jax
import jax.numpy as jnp
from jax.experimental import pallas as pl
from jax.experimental.pallas import tpu as pltpu

_BM = 400


def _inner(adj_blk, x_blk, out_blk):
    out_blk[...] = jnp.dot(adj_blk[...].astype(jnp.bfloat16),
                           x_blk[...].astype(jnp.bfloat16),
                           preferred_element_type=jnp.float32)


def kernel(adj, x):
    m, k = adj.shape
    _, n = x.shape
    bm = _BM if m % _BM == 0 else m

    def outer(adj_hbm, x_hbm, out_hbm):
        pipe = pltpu.emit_pipeline(
            _inner,
            grid=(m // bm,),
            in_specs=[
                pl.BlockSpec((bm, k), lambda i: (i, 0),
                             pipeline_mode=pl.Buffered(buffer_count=3)),
                pl.BlockSpec((k, n), lambda i: (0, 0)),
            ],
            out_specs=[
                pl.BlockSpec((bm, n), lambda i: (i, 0)),
            ],
        )
        pipe(adj_hbm, x_hbm, out_hbm)

    return pl.pallas_call(
        outer,
        in_specs=[
            pl.BlockSpec(memory_space=pl.ANY),
            pl.BlockSpec(memory_space=pl.ANY),
        ],
        out_specs=pl.BlockSpec(memory_space=pl.ANY),
        out_shape=jax.ShapeDtypeStruct((m, n), jnp.float32),
    )(adj, x)


# speedup vs baseline: 1.0396x; 1.0396x over previous
import jax
import jax.numpy as jnp
from jax.experimental import pallas as pl
from jax.experimental.pallas import tpu as pltpu

_BM = 400


def _mm_kernel(adj_ref, x_ref, out_ref, xb_ref):
    @pl.when(pl.program_id(0) == 0)
    def _():
        xb_ref[...] = x_ref[...].astype(jnp.bfloat16)

    out_ref[...] = jnp.dot(adj_ref[...].astype(jnp.bfloat16), xb_ref[...],
                           preferred_element_type=jnp.float32)


def kernel(adj, x):
    m, k = adj.shape
    _, n = x.shape
    bm = _BM if m % _BM == 0 else m
    return pl.pallas_call(
        _mm_kernel,
        grid=(m // bm,),
        in_specs=[
            pl.BlockSpec((bm, k), lambda i: (i, 0)),
            pl.BlockSpec((k, n), lambda i: (0, 0)),
        ],
        out_specs=pl.BlockSpec((bm, n), lambda i: (i, 0)),
        out_shape=jax.ShapeDtypeStruct((m, n), jnp.float32),
        scratch_shapes=[pltpu.VMEM((k, n), jnp.bfloat16)],
        compiler_params=pltpu.CompilerParams(
            dimension_semantics=("arbitrary",),
        ),
    )(adj, x)
